# baseline (speedup 1.0000x reference)
import jax
import jax.numpy as jnp
from jax import lax
from jax.experimental import pallas as pl
from jax.experimental.pallas import tpu as pltpu

B, G, D = 16, 2048, 128
KS = 2
GK = G // KS


def _tc_body(phone_ref, adj_grp_ref, h_ref, obf_ref, oaf_ref):
    i = pl.program_id(0)
    k = pl.program_id(1)
    p = phone_ref[i]
    off = p % 8
    grp = adj_grp_ref[...]
    rsel = lax.broadcasted_iota(jnp.int32, (8, GK), 0)
    row = jnp.sum(jnp.where(rsel == off, grp, 0.0), axis=0, keepdims=True)
    j = lax.broadcasted_iota(jnp.int32, (1, GK), 1) + k * GK
    wbf = jnp.where(j < p, row, 0.0)
    waf = jnp.where(j > p, row, 0.0)
    w = jnp.concatenate([wbf, waf], axis=0)
    r = lax.dot_general(w, h_ref[0], (((1,), (0,)), ((), ())),
                        preferred_element_type=jnp.float32)

    @pl.when(k == 0)
    def _():
        obf_ref[0, 0] = r[0]
        oaf_ref[0, 0] = r[1]

    @pl.when(k != 0)
    def _():
        obf_ref[0, 0] = obf_ref[0, 0] + r[0]
        oaf_ref[0, 0] = oaf_ref[0, 0] + r[1]


def kernel(h, adj_c, phone_set):
    phone = phone_set.astype(jnp.int32)

    grid_spec = pltpu.PrefetchScalarGridSpec(
        num_scalar_prefetch=1,
        grid=(B, KS),
        in_specs=[
            pl.BlockSpec((8, GK),
                         lambda i, k, ph: (i * (G // 8) + ph[i] // 8, k)),
            pl.BlockSpec((1, GK, D), lambda i, k, ph: (i, k, 0)),
        ],
        out_specs=[
            pl.BlockSpec((1, 1, D), lambda i, k, ph: (i, 0, 0)),
            pl.BlockSpec((1, 1, D), lambda i, k, ph: (i, 0, 0)),
        ],
    )
    g_bf, g_af = pl.pallas_call(
        _tc_body,
        grid_spec=grid_spec,
        compiler_params=pltpu.CompilerParams(
            dimension_semantics=("parallel", "arbitrary")),
        out_shape=(
            jax.ShapeDtypeStruct((B, 1, D), jnp.float32),
            jax.ShapeDtypeStruct((B, 1, D), jnp.float32),
        ),
    )(phone, adj_c.reshape(B * G, G), h)
    return (g_bf.reshape(B, D), g_af.reshape(B, D))


# speedup vs baseline: 1.5051x; 1.5051x over previous
import jax
import jax.numpy as jnp
from jax import lax
from jax.experimental import pallas as pl
from jax.experimental.pallas import tpu as pltpu

B, G, D = 16, 2048, 128


def _row_copy(adj_hbm, rows_v, sems, b, pb):
    return pltpu.make_async_copy(
        adj_hbm.at[pl.ds(b * G + pb, 1), :],
        rows_v.at[pl.ds(b, 1), :],
        sems.at[b])


def _tc_body(phone_ref, adj_hbm, h_ref, obf_ref, oaf_ref, rows_v, sems):
    i = pl.program_id(0)

    @pl.when(i == 0)
    def _():
        for b in range(B):
            _row_copy(adj_hbm, rows_v, sems, b, phone_ref[b]).start()

    p = phone_ref[i]
    _row_copy(adj_hbm, rows_v, sems, i, p).wait()

    row = rows_v[pl.ds(i, 1), :]
    j = lax.broadcasted_iota(jnp.int32, (1, G), 1)
    wbf = jnp.where(j < p, row, 0.0)
    waf = jnp.where(j > p, row, 0.0)
    w = jnp.concatenate([wbf, waf], axis=0)
    r = lax.dot_general(w, h_ref[0], (((1,), (0,)), ((), ())),
                        preferred_element_type=jnp.float32)
    obf_ref[0, 0] = r[0]
    oaf_ref[0, 0] = r[1]


def kernel(h, adj_c, phone_set):
    phone = phone_set.astype(jnp.int32)

    grid_spec = pltpu.PrefetchScalarGridSpec(
        num_scalar_prefetch=1,
        grid=(B,),
        in_specs=[
            pl.BlockSpec(memory_space=pltpu.MemorySpace.HBM),
            pl.BlockSpec((1, G, D), lambda i, ph: (i, 0, 0)),
        ],
        out_specs=[
            pl.BlockSpec((1, 1, D), lambda i, ph: (i, 0, 0)),
            pl.BlockSpec((1, 1, D), lambda i, ph: (i, 0, 0)),
        ],
        scratch_shapes=[
            pltpu.VMEM((B, G), jnp.float32),
            pltpu.SemaphoreType.DMA((B,)),
        ],
    )
    g_bf, g_af = pl.pallas_call(
        _tc_body,
        grid_spec=grid_spec,
        compiler_params=pltpu.CompilerParams(
            dimension_semantics=("arbitrary",)),
        out_shape=(
            jax.ShapeDtypeStruct((B, 1, D), jnp.float32),
            jax.ShapeDtypeStruct((B, 1, D), jnp.float32),
        ),
    )(phone, adj_c.reshape(B * G, G), h)
    return (g_bf.reshape(B, D), g_af.reshape(B, D))


# speedup vs baseline: 2.1706x; 1.4422x over previous
import jax
import jax.numpy as jnp
from jax import lax
from jax.experimental import pallas as pl
from jax.experimental.pallas import tpu as pltpu

B, G, D = 16, 2048, 128


def _tc_body(phone_ref, adjA_ref, adjB_ref, h_ref, obf_ref, oaf_ref):
    i = pl.program_id(0)

    def one(b_local, adj_ref, p):
        off = p % 8
        grp = adj_ref[...]
        rsel = lax.broadcasted_iota(jnp.int32, (8, G), 0)
        row = jnp.sum(jnp.where(rsel == off, grp, 0.0), axis=0, keepdims=True)
        j = lax.broadcasted_iota(jnp.int32, (1, G), 1)
        wbf = jnp.where(j < p, row, 0.0)
        waf = jnp.where(j > p, row, 0.0)
        w = jnp.concatenate([wbf, waf], axis=0)
        r = lax.dot_general(w, h_ref[b_local], (((1,), (0,)), ((), ())),
                            preferred_element_type=jnp.float32)
        obf_ref[b_local, 0] = r[0]
        oaf_ref[b_local, 0] = r[1]

    one(0, adjA_ref, phone_ref[2 * i])
    one(1, adjB_ref, phone_ref[2 * i + 1])


def kernel(h, adj_c, phone_set):
    phone = phone_set.astype(jnp.int32)

    grid_spec = pltpu.PrefetchScalarGridSpec(
        num_scalar_prefetch=1,
        grid=(B // 2,),
        in_specs=[
            pl.BlockSpec((8, G),
                         lambda i, ph: (2 * i * (G // 8) + ph[2 * i] // 8, 0)),
            pl.BlockSpec((8, G),
                         lambda i, ph: ((2 * i + 1) * (G // 8)
                                        + ph[2 * i + 1] // 8, 0)),
            pl.BlockSpec((2, G, D), lambda i, ph: (i, 0, 0)),
        ],
        out_specs=[
            pl.BlockSpec((2, 1, D), lambda i, ph: (i, 0, 0)),
            pl.BlockSpec((2, 1, D), lambda i, ph: (i, 0, 0)),
        ],
    )
    call = pl.pallas_call(
        _tc_body,
        grid_spec=grid_spec,
        compiler_params=pltpu.CompilerParams(
            dimension_semantics=("parallel",)),
        out_shape=(
            jax.ShapeDtypeStruct((B, 1, D), jnp.float32),
            jax.ShapeDtypeStruct((B, 1, D), jnp.float32),
        ),
    )
    adj2 = adj_c.reshape(B * G, G)
    g_bf, g_af = call(phone, adj2, adj2, h)
    return (g_bf.reshape(B, D), g_af.reshape(B, D))


# speedup vs baseline: 2.6123x; 1.2035x over previous
import functools

import jax
import jax.numpy as jnp
from jax import lax
from jax.experimental import pallas as pl
from jax.experimental.pallas import tpu as pltpu

B, G, D = 16, 2048, 128
NB = 4


def _tc_body(phone_ref, *refs):
    adj_refs = refs[:NB]
    h_ref, obf_ref, oaf_ref = refs[NB:]
    i = pl.program_id(0)

    for bl in range(NB):
        p = phone_ref[NB * i + bl]
        off = p % 8
        grp = adj_refs[bl][...]
        rsel = lax.broadcasted_iota(jnp.int32, (8, G), 0)
        row = jnp.sum(jnp.where(rsel == off, grp, 0.0), axis=0, keepdims=True)
        j = lax.broadcasted_iota(jnp.int32, (1, G), 1)
        wbf = jnp.where(j < p, row, 0.0)
        waf = jnp.where(j > p, row, 0.0)
        w = jnp.concatenate([wbf, waf], axis=0)
        r = lax.dot_general(w, h_ref[bl], (((1,), (0,)), ((), ())),
                            preferred_element_type=jnp.float32)
        obf_ref[bl, 0] = r[0]
        oaf_ref[bl, 0] = r[1]


def _adj_index_map(bl, i, ph):
    return ((NB * i + bl) * (G // 8) + ph[NB * i + bl] // 8, 0)


def kernel(h, adj_c, phone_set):
    phone = phone_set.astype(jnp.int32)

    grid_spec = pltpu.PrefetchScalarGridSpec(
        num_scalar_prefetch=1,
        grid=(B // NB,),
        in_specs=[
            pl.BlockSpec((8, G), functools.partial(_adj_index_map, bl))
            for bl in range(NB)
        ] + [
            pl.BlockSpec((NB, G, D), lambda i, ph: (i, 0, 0)),
        ],
        out_specs=[
            pl.BlockSpec((NB, 1, D), lambda i, ph: (i, 0, 0)),
            pl.BlockSpec((NB, 1, D), lambda i, ph: (i, 0, 0)),
        ],
    )
    call = pl.pallas_call(
        _tc_body,
        grid_spec=grid_spec,
        compiler_params=pltpu.CompilerParams(
            dimension_semantics=("parallel",)),
        out_shape=(
            jax.ShapeDtypeStruct((B, 1, D), jnp.float32),
            jax.ShapeDtypeStruct((B, 1, D), jnp.float32),
        ),
    )
    adj2 = adj_c.reshape(B * G, G)
    g_bf, g_af = call(phone, *([adj2] * NB), h)
    return (g_bf.reshape(B, D), g_af.reshape(B, D))


# speedup vs baseline: 2.6163x; 1.0016x over previous
import functools

import jax
import jax.numpy as jnp
from jax import lax
from jax.experimental import pallas as pl
from jax.experimental.pallas import tpu as pltpu

B, G, D = 16, 2048, 128
NB = 8


def _tc_body(phone_ref, *refs):
    adj_refs = refs[:NB]
    h_ref, obf_ref, oaf_ref = refs[NB:]
    i = pl.program_id(0)

    for bl in range(NB):
        p = phone_ref[NB * i + bl]
        off = p % 8
        grp = adj_refs[bl][...]
        rsel = lax.broadcasted_iota(jnp.int32, (8, G), 0)
        row = jnp.sum(jnp.where(rsel == off, grp, 0.0), axis=0, keepdims=True)
        j = lax.broadcasted_iota(jnp.int32, (1, G), 1)
        wbf = jnp.where(j < p, row, 0.0)
        waf = jnp.where(j > p, row, 0.0)
        w = jnp.concatenate([wbf, waf], axis=0)
        r = lax.dot_general(w, h_ref[bl], (((1,), (0,)), ((), ())),
                            preferred_element_type=jnp.float32)
        obf_ref[bl, 0] = r[0]
        oaf_ref[bl, 0] = r[1]


def _adj_index_map(bl, i, ph):
    return ((NB * i + bl) * (G // 8) + ph[NB * i + bl] // 8, 0)


def kernel(h, adj_c, phone_set):
    phone = phone_set.astype(jnp.int32)

    grid_spec = pltpu.PrefetchScalarGridSpec(
        num_scalar_prefetch=1,
        grid=(B // NB,),
        in_specs=[
            pl.BlockSpec((8, G), functools.partial(_adj_index_map, bl))
            for bl in range(NB)
        ] + [
            pl.BlockSpec((NB, G, D), lambda i, ph: (i, 0, 0)),
        ],
        out_specs=[
            pl.BlockSpec((NB, 1, D), lambda i, ph: (i, 0, 0)),
            pl.BlockSpec((NB, 1, D), lambda i, ph: (i, 0, 0)),
        ],
    )
    call = pl.pallas_call(
        _tc_body,
        grid_spec=grid_spec,
        compiler_params=pltpu.CompilerParams(
            dimension_semantics=("parallel",)),
        out_shape=(
            jax.ShapeDtypeStruct((B, 1, D), jnp.float32),
            jax.ShapeDtypeStruct((B, 1, D), jnp.float32),
        ),
    )
    adj2 = adj_c.reshape(B * G, G)
    g_bf, g_af = call(phone, *([adj2] * NB), h)
    return (g_bf.reshape(B, D), g_af.reshape(B, D))


# speedup vs baseline: 2.7722x; 1.0596x over previous
import functools

import jax
import jax.numpy as jnp
from jax import lax
from jax.experimental import pallas as pl
from jax.experimental.pallas import tpu as pltpu

B, G, D = 16, 2048, 128
NB = 8
NH = 4
HB = NB // NH


def _tc_body(phone_ref, *refs):
    adj_refs = refs[:NB]
    h_refs = refs[NB:NB + NH]
    obf_ref, oaf_ref = refs[NB + NH:]
    i = pl.program_id(0)

    for bl in range(NB):
        p = phone_ref[NB * i + bl]
        off = p % 8
        grp = adj_refs[bl][...]
        rsel = lax.broadcasted_iota(jnp.int32, (8, G), 0)
        row = jnp.sum(jnp.where(rsel == off, grp, 0.0), axis=0, keepdims=True)
        j = lax.broadcasted_iota(jnp.int32, (1, G), 1)
        wbf = jnp.where(j < p, row, 0.0)
        waf = jnp.where(j > p, row, 0.0)
        w = jnp.concatenate([wbf, waf], axis=0)
        hmat = h_refs[bl // HB][bl % HB]
        r = lax.dot_general(w, hmat, (((1,), (0,)), ((), ())),
                            preferred_element_type=jnp.float32)
        obf_ref[bl, 0] = r[0]
        oaf_ref[bl, 0] = r[1]


def _adj_index_map(bl, i, ph):
    return ((NB * i + bl) * (G // 8) + ph[NB * i + bl] // 8, 0)


def _h_index_map(j, i, ph):
    return (i * NH + j, 0, 0)


def kernel(h, adj_c, phone_set):
    phone = phone_set.astype(jnp.int32)

    grid_spec = pltpu.PrefetchScalarGridSpec(
        num_scalar_prefetch=1,
        grid=(B // NB,),
        in_specs=[
            pl.BlockSpec((8, G), functools.partial(_adj_index_map, bl))
            for bl in range(NB)
        ] + [
            pl.BlockSpec((HB, G, D), functools.partial(_h_index_map, j))
            for j in range(NH)
        ],
        out_specs=[
            pl.BlockSpec((NB, 1, D), lambda i, ph: (i, 0, 0)),
            pl.BlockSpec((NB, 1, D), lambda i, ph: (i, 0, 0)),
        ],
    )
    call = pl.pallas_call(
        _tc_body,
        grid_spec=grid_spec,
        compiler_params=pltpu.CompilerParams(
            dimension_semantics=("parallel",)),
        out_shape=(
            jax.ShapeDtypeStruct((B, 1, D), jnp.float32),
            jax.ShapeDtypeStruct((B, 1, D), jnp.float32),
        ),
    )
    adj2 = adj_c.reshape(B * G, G)
    g_bf, g_af = call(phone, *([adj2] * NB), *([h] * NH))
    return (g_bf.reshape(B, D), g_af.reshape(B, D))
